# Initial kernel scaffold; baseline (speedup 1.0000x reference)
import jax
import jax.numpy as jnp
from jax.experimental import pallas as pl


def kernel(reg_pred, targets, num_level_bboxes, cls_pred):
    raise NotImplementedError("write your pallas kernel here")


# speedup vs baseline: 2.1303x; 2.1303x over previous
import functools

import jax
import jax.numpy as jnp
from jax.experimental import pallas as pl
from jax.experimental.pallas import tpu as pltpu


def _body(reg_ref, tgt_ref, cls_ref, asg_ref, dis_ref, lbl_ref):
    reg = reg_ref[...]
    tgt = tgt_ref[...]
    cls = cls_ref[...]
    G = tgt.shape[0]
    C = cls.shape[1]

    cx = (reg[:, 0] + reg[:, 2]) / 2.0
    cy = (reg[:, 1] + reg[:, 3]) / 2.0
    gx = ((tgt[:, 0] + tgt[:, 2]) / 2.0).reshape(1, G)
    gy = ((tgt[:, 1] + tgt[:, 3]) / 2.0).reshape(1, G)
    glb = tgt[:, 4].reshape(1, G)

    dx = cx[:, None] - gx
    dy = cy[:, None] - gy
    dist = jnp.sqrt(dx * dx + dy * dy)

    min_d = jnp.min(dist, axis=1)
    giota = jax.lax.broadcasted_iota(jnp.int32, dist.shape, 1)
    idx = jnp.min(jnp.where(dist == min_d[:, None], giota, G), axis=1)

    onehot = giota == idx[:, None]
    glabel = jnp.sum(jnp.where(onehot, glb, 0.0), axis=1)

    maxv = jnp.max(cls, axis=1)
    ciota = jax.lax.broadcasted_iota(jnp.int32, cls.shape, 1)
    cidx = jnp.min(jnp.where(cls == maxv[:, None], ciota, C), axis=1)

    pos = (maxv > 0.0) & (cidx == glabel.astype(jnp.int32))
    asg_ref[0, 0, :] = jnp.where(pos, idx + 1, 0)
    dis_ref[0, 0, :] = min_d
    lbl_ref[0, 0, :] = jnp.where(pos, glabel.astype(jnp.int32), -1)


@functools.partial(jax.jit, static_argnames=())
def _run(reg_pred, targets, cls_pred):
    N = reg_pred.shape[0]
    G = targets.shape[0]
    C = cls_pred.shape[1]
    B = 1000
    NB = N // B

    grid = (NB,)
    out_shapes = (
        jax.ShapeDtypeStruct((NB, 1, B), jnp.int32),
        jax.ShapeDtypeStruct((NB, 1, B), jnp.float32),
        jax.ShapeDtypeStruct((NB, 1, B), jnp.int32),
    )
    in_specs = [
        pl.BlockSpec((B, 4), lambda i: (i, 0)),
        pl.BlockSpec((G, 5), lambda i: (0, 0)),
        pl.BlockSpec((B, C), lambda i: (i, 0)),
    ]
    out_specs = (
        pl.BlockSpec((1, 1, B), lambda i: (i, 0, 0)),
        pl.BlockSpec((1, 1, B), lambda i: (i, 0, 0)),
        pl.BlockSpec((1, 1, B), lambda i: (i, 0, 0)),
    )
    asg, dis, lbl = pl.pallas_call(
        _body,
        grid=grid,
        in_specs=in_specs,
        out_specs=out_specs,
        out_shape=out_shapes,
    )(reg_pred, targets, cls_pred)
    return asg.reshape(N), dis.reshape(N), lbl.reshape(N)


def kernel(reg_pred, targets, num_level_bboxes, cls_pred):
    asg, dis, lbl = _run(reg_pred, targets, cls_pred)
    return (asg, dis, lbl, reg_pred, targets)
